# Initial kernel scaffold; baseline (speedup 1.0000x reference)
import jax
import jax.numpy as jnp
from jax.experimental import pallas as pl


def kernel(x):
    raise NotImplementedError("write your pallas kernel here")


# speedup vs baseline: 38.1607x; 38.1607x over previous
import functools
import math

import jax
import jax.numpy as jnp
from jax.experimental import pallas as pl
from jax.experimental.pallas import tpu as pltpu

_ROWS_PER_BLOCK = 16


def _topk_mask_body(k, x_ref, o_ref):
    x = x_ref[...]
    xb = jax.lax.bitcast_convert_type(x, jnp.int32)
    s = xb ^ (jax.lax.shift_right_arithmetic(xb, 31) & jnp.int32(0x7FFFFFFF))
    rows = x.shape[0]
    int_min = jnp.int32(-2147483648)

    def step(i, prefix_u):
        bit = jax.lax.shift_left(jnp.int32(1), 31 - i)
        trial_u = prefix_u | bit
        trial_s = trial_u ^ int_min
        cmp = s >= trial_s
        cnt = jnp.sum(jnp.where(cmp, jnp.int32(1), jnp.int32(0)), axis=1,
                      keepdims=True)
        return jnp.where(cnt >= k, trial_u, prefix_u)

    prefix_u = jax.lax.fori_loop(
        0, 32, step, jnp.zeros((rows, 1), jnp.int32), unroll=True)
    thr_s = prefix_u ^ int_min
    o_ref[...] = jnp.where(s >= thr_s, x, jnp.float32(0.0))


def kernel(x):
    n, d = x.shape
    k = max(1, int(math.ceil(0.1 * d)))
    rb = _ROWS_PER_BLOCK
    grid = (n // rb,)
    return pl.pallas_call(
        functools.partial(_topk_mask_body, k),
        grid=grid,
        in_specs=[pl.BlockSpec((rb, d), lambda i: (i, 0))],
        out_specs=pl.BlockSpec((rb, d), lambda i: (i, 0)),
        out_shape=jax.ShapeDtypeStruct((n, d), x.dtype),
        compiler_params=pltpu.CompilerParams(
            dimension_semantics=("arbitrary",)),
    )(x)
